# Initial kernel scaffold; baseline (speedup 1.0000x reference)
import jax
import jax.numpy as jnp
from jax.experimental import pallas as pl


def kernel(x, edge_index, params):
    raise NotImplementedError("write your pallas kernel here")


# speedup vs baseline: 1.0125x; 1.0125x over previous
import functools
import jax
import jax.numpy as jnp
from jax import lax
from jax.experimental import pallas as pl
from jax.experimental.pallas import tpu as pltpu
from jax.experimental.pallas import tpu_sc as plsc

N0V, EV, HV = 10000, 320000, 128
N1V, N2V = 8000, 4800
NC, NS = 2, 16
NW = NC * NS
KE = 128
ECH = -(-EV // (NW * KE))
EPAD = NW * ECH * KE
DUMMY = N0V
NPAD = N0V + 112
RPT = NPAD // NS
GCH = 2
GPAD = NW * GCH * KE
RB = 400


@functools.cache
def _mesh():
    return plsc.VectorSubcoreMesh(core_axis_name="c", subcore_axis_name="s",
                                  num_cores=NC, num_subcores=NS)


def _sc_agg_body(y_hbm, src_hbm, dst_hbm, z_hbm, out_hbm,
                 src_v, dst_v, rows_v, sem, acc):
    cid = lax.axis_index("c")
    sid = lax.axis_index("s")
    gwid = cid * NS + sid
    pltpu.sync_copy(src_hbm.at[gwid], src_v)
    pltpu.sync_copy(dst_hbm.at[gwid], dst_v)
    r0 = sid * RPT
    pltpu.sync_copy(z_hbm.at[pl.ds(r0, RPT)], acc.at[pl.ds(r0, RPT)])
    plsc.subcore_barrier()

    def chunk(j, carry):
        pltpu.async_copy(y_hbm.at[src_v.at[j]], rows_v, sem).wait()
        pltpu.sync_copy(rows_v, acc.at[dst_v.at[j]], add=True)
        return carry

    lax.fori_loop(0, ECH, chunk, 0)
    plsc.subcore_barrier()
    pltpu.sync_copy(acc.at[pl.ds(r0, RPT)], out_hbm.at[cid, pl.ds(r0, RPT)])


def _sc_aggregate(y, srcA, dstA, z128):
    return pl.kernel(
        _sc_agg_body,
        out_type=jax.ShapeDtypeStruct((NC, NPAD, HV), jnp.float32),
        mesh=_mesh(),
        scratch_types=[
            pltpu.VMEM((ECH, KE), jnp.int32),
            pltpu.VMEM((ECH, KE), jnp.int32),
            pltpu.VMEM((KE, HV), jnp.float32),
            pltpu.SemaphoreType.DMA,
            pltpu.VMEM_SHARED((NPAD, HV), jnp.float32),
        ],
    )(y, srcA, dstA, z128)


def _sc_gather_body(tab_hbm, idx_hbm, out_hbm, idx_v, rows_v, sem):
    cid = lax.axis_index("c")
    sid = lax.axis_index("s")
    gwid = cid * NS + sid
    pltpu.sync_copy(idx_hbm.at[gwid], idx_v)
    for j in range(GCH):
        pltpu.async_copy(tab_hbm.at[idx_v.at[j]], rows_v, sem).wait()
        pltpu.sync_copy(rows_v, out_hbm.at[pl.ds((gwid * GCH + j) * KE, KE)])


def _sc_gather_rows(tab, idxA):
    return pl.kernel(
        _sc_gather_body,
        out_type=jax.ShapeDtypeStruct((GPAD, HV), jnp.float32),
        mesh=_mesh(),
        scratch_types=[
            pltpu.VMEM((GCH, KE), jnp.int32),
            pltpu.VMEM((KE, HV), jnp.float32),
            pltpu.SemaphoreType.DMA,
        ],
    )(tab, idxA)


def _gather_pad(tab, perm):
    idx = jnp.concatenate(
        [perm, jnp.zeros((GPAD - perm.shape[0],), jnp.int32)])
    return _sc_gather_rows(tab, idx.reshape(NW, GCH, KE))[:perm.shape[0]]


def _mm_body(x_ref, w_ref, b_ref, o_ref, *, act):
    h = jnp.dot(x_ref[...], w_ref[...], preferred_element_type=jnp.float32)
    h = h + b_ref[...]
    o_ref[...] = jnp.maximum(h, 0.0) if act else h


def _tc_mm(x, w, b, act):
    n, din = x.shape
    dout = w.shape[1]
    return pl.pallas_call(
        functools.partial(_mm_body, act=act),
        grid=(n // RB,),
        in_specs=[
            pl.BlockSpec((RB, din), lambda i: (i, 0)),
            pl.BlockSpec((din, dout), lambda i: (0, 0)),
            pl.BlockSpec((1, dout), lambda i: (0, 0)),
        ],
        out_specs=pl.BlockSpec((RB, dout), lambda i: (i, 0)),
        out_shape=jax.ShapeDtypeStruct((n, dout), jnp.float32),
    )(x, w, b.reshape(1, dout))


def _mmdual_body(x_ref, w_ref, dis_ref, xw_ref, y_ref):
    xw = jnp.dot(x_ref[...], w_ref[...], preferred_element_type=jnp.float32)
    xw_ref[...] = xw
    y_ref[...] = xw * dis_ref[...]


def _tc_mmdual(x, w, dis):
    n = x.shape[0]
    return pl.pallas_call(
        _mmdual_body,
        grid=(n // RB,),
        in_specs=[
            pl.BlockSpec((RB, HV), lambda i: (i, 0)),
            pl.BlockSpec((HV, HV), lambda i: (0, 0)),
            pl.BlockSpec((RB, 1), lambda i: (i, 0)),
        ],
        out_specs=[
            pl.BlockSpec((RB, HV), lambda i: (i, 0)),
            pl.BlockSpec((RB, HV), lambda i: (i, 0)),
        ],
        out_shape=[
            jax.ShapeDtypeStruct((n, HV), jnp.float32),
            jax.ShapeDtypeStruct((n, HV), jnp.float32),
        ],
    )(x, w, dis)


def _mmdual2_body(u_ref, k_ref, wt_ref, wb_ref, dis_ref, xw_ref, y_ref):
    xw = jnp.dot(u_ref[...], wt_ref[...], preferred_element_type=jnp.float32)
    xw = xw + jnp.dot(k_ref[...], wb_ref[...], preferred_element_type=jnp.float32)
    xw_ref[...] = xw
    y_ref[...] = xw * dis_ref[...]


def _tc_mmdual2(u, skip, wt, wb, dis):
    n = u.shape[0]
    return pl.pallas_call(
        _mmdual2_body,
        grid=(n // RB,),
        in_specs=[
            pl.BlockSpec((RB, HV), lambda i: (i, 0)),
            pl.BlockSpec((RB, HV), lambda i: (i, 0)),
            pl.BlockSpec((HV, HV), lambda i: (0, 0)),
            pl.BlockSpec((HV, HV), lambda i: (0, 0)),
            pl.BlockSpec((RB, 1), lambda i: (i, 0)),
        ],
        out_specs=[
            pl.BlockSpec((RB, HV), lambda i: (i, 0)),
            pl.BlockSpec((RB, HV), lambda i: (i, 0)),
        ],
        out_shape=[
            jax.ShapeDtypeStruct((n, HV), jnp.float32),
            jax.ShapeDtypeStruct((n, HV), jnp.float32),
        ],
    )(u, skip, wt, wb, dis)


def _combine_body(aggp_ref, xw_ref, dis_ref, b_ref, h_ref, s_ref, q_ref):
    agg = aggp_ref[0] + aggp_ref[1]
    dis = dis_ref[...]
    h = agg * dis + xw_ref[...] * (dis * dis) + b_ref[...]
    h_ref[...] = h

    @pl.when(pl.program_id(0) == 0)
    def _init():
        s_ref[...] = jnp.zeros_like(s_ref)
        q_ref[...] = jnp.zeros_like(q_ref)

    s_ref[...] += jnp.sum(h, axis=0, keepdims=True)
    q_ref[...] += jnp.sum(h * h, axis=0, keepdims=True)


def _tc_combine(aggp, xw, dis, b):
    n = xw.shape[0]
    return pl.pallas_call(
        _combine_body,
        grid=(n // RB,),
        in_specs=[
            pl.BlockSpec((NC, RB, HV), lambda i: (0, i, 0)),
            pl.BlockSpec((RB, HV), lambda i: (i, 0)),
            pl.BlockSpec((RB, 1), lambda i: (i, 0)),
            pl.BlockSpec((1, HV), lambda i: (0, 0)),
        ],
        out_specs=[
            pl.BlockSpec((RB, HV), lambda i: (i, 0)),
            pl.BlockSpec((1, HV), lambda i: (0, 0)),
            pl.BlockSpec((1, HV), lambda i: (0, 0)),
        ],
        out_shape=[
            jax.ShapeDtypeStruct((n, HV), jnp.float32),
            jax.ShapeDtypeStruct((1, HV), jnp.float32),
            jax.ShapeDtypeStruct((1, HV), jnp.float32),
        ],
    )(aggp, xw, dis, b.reshape(1, HV))


def _bn_body(h_ref, s_ref, q_ref, g_ref, be_ref, o_ref, *, n):
    m = s_ref[...] * (1.0 / n)
    v = q_ref[...] * (1.0 / n) - m * m
    o_ref[...] = jnp.maximum(
        (h_ref[...] - m) * lax.rsqrt(v + 1e-5) * g_ref[...] + be_ref[...], 0.0)


def _tc_bn_relu(h, s, q, g, be):
    n = h.shape[0]
    return pl.pallas_call(
        functools.partial(_bn_body, n=float(n)),
        grid=(n // RB,),
        in_specs=[
            pl.BlockSpec((RB, HV), lambda i: (i, 0)),
            pl.BlockSpec((1, HV), lambda i: (0, 0)),
            pl.BlockSpec((1, HV), lambda i: (0, 0)),
            pl.BlockSpec((1, HV), lambda i: (0, 0)),
            pl.BlockSpec((1, HV), lambda i: (0, 0)),
        ],
        out_specs=pl.BlockSpec((RB, HV), lambda i: (i, 0)),
        out_shape=jax.ShapeDtypeStruct((n, HV), jnp.float32),
    )(h, s, q, g.reshape(1, HV), be.reshape(1, HV))


def _ref_gcn(x, src, dst, valid, n, W, b):
    xw = x @ W
    vf = valid.astype(x.dtype)
    deg = jax.ops.segment_sum(vf, dst, num_segments=n) + 1.0
    dis = lax.rsqrt(deg)
    norm = dis[src] * dis[dst] * vf
    agg = jax.ops.segment_sum(xw[src] * norm[:, None], dst, num_segments=n)
    return agg + xw * (dis * dis)[:, None] + b


def _ref_bn(h, g, b):
    m = h.mean(axis=0)
    v = ((h - m) ** 2).mean(axis=0)
    return (h - m) * lax.rsqrt(v + 1e-5) * g + b


def _ref_block(x, src, dst, valid, n, W, b, g, be):
    return jax.nn.relu(_ref_bn(_ref_gcn(x, src, dst, valid, n, W, b), g, be))


def _pallas_block(xw_y, srcA, dstA_eff, dis, b, g, be):
    xw, y = xw_y
    z128 = jnp.zeros((NPAD, HV), jnp.float32)
    aggp = _sc_aggregate(y, srcA, dstA_eff, z128)
    h, s, q = _tc_combine(aggp, xw, dis, b)
    return _tc_bn_relu(h, s, q, g, be)


def _pad_edges(src, dst, valid):
    pad = EPAD - EV
    srcP = jnp.concatenate([src, jnp.zeros((pad,), jnp.int32)])
    dstP = jnp.concatenate([jnp.where(valid, dst, DUMMY),
                            jnp.full((pad,), DUMMY, jnp.int32)])
    return srcP.reshape(NW, ECH, KE), dstP.reshape(NW, ECH, KE)


def _dis_of(valid, dst, n):
    deg = jax.ops.segment_sum(valid.astype(jnp.float32), dst, num_segments=n) + 1.0
    return lax.rsqrt(deg)[:, None]


def kernel(x, edge_index, params):
    p = params
    src0, dst0 = edge_index[0], edge_index[1]
    valid0 = jnp.ones((EV,), bool)

    x0 = jax.nn.relu(x @ p['W_in'] + p['b_in'])
    d0 = _ref_block(x0, src0, dst0, valid0, N0V,
                    p['W_d0'], p['b_d0'], p['g_d0'], p['be_d0'])
    score0 = jnp.tanh((d0 @ p['p0']) / jnp.linalg.norm(p['p0']))
    vals0, perm0 = lax.top_k(score0, N1V)
    x1 = _gather_pad(d0, perm0) * vals0[:, None]
    keep0 = jnp.zeros((N0V,), bool).at[perm0].set(True)
    nid0 = jnp.zeros((N0V,), jnp.int32).at[perm0].set(
        jnp.arange(N1V, dtype=jnp.int32))
    valid1 = valid0 & keep0[src0] & keep0[dst0]
    src1 = jnp.where(valid1, nid0[src0], 0)
    dst1 = jnp.where(valid1, nid0[dst0], 0)

    d1 = _ref_block(x1, src1, dst1, valid1, N1V,
                    p['W_d1'], p['b_d1'], p['g_d1'], p['be_d1'])
    score1 = jnp.tanh((d1 @ p['p1']) / jnp.linalg.norm(p['p1']))
    vals1, perm1 = lax.top_k(score1, N2V)
    x2 = _gather_pad(d1, perm1) * vals1[:, None]
    keep1 = jnp.zeros((N1V,), bool).at[perm1].set(True)
    nid1 = jnp.zeros((N1V,), jnp.int32).at[perm1].set(
        jnp.arange(N2V, dtype=jnp.int32))
    valid2 = valid1 & keep1[src1] & keep1[dst1]
    src2 = jnp.where(valid2, nid1[src1], 0)
    dst2 = jnp.where(valid2, nid1[dst1], 0)

    srcA2, dstA2 = _pad_edges(src2, dst2, valid2)
    dis2 = _dis_of(valid2, dst2, N2V)
    xw_y = _tc_mmdual(x2, p['W_bt'], dis2)
    bt = _pallas_block(xw_y, srcA2, dstA2, dis2,
                       p['b_bt'], p['g_bt'], p['be_bt'])

    srcA1, dstA1 = _pad_edges(src1, dst1, valid1)
    dis1f = _dis_of(valid1, dst1, N0V)
    u0 = jnp.concatenate([bt, jnp.zeros((N0V - N2V, HV), jnp.float32)])
    xw_y = _tc_mmdual2(u0, d0, p['W_u0'][:HV], p['W_u0'][HV:], dis1f)
    h = _pallas_block(xw_y, srcA1, dstA1, dis1f,
                      p['b_u0'], p['g_u0'], p['be_u0'])

    srcA0, dstA0 = _pad_edges(src0, dst0, valid0)
    dis0 = _dis_of(valid0, dst0, N0V)
    u1 = jnp.concatenate([h[:N1V], jnp.zeros((N0V - N1V, HV), jnp.float32)])
    xw_y = _tc_mmdual2(u1, x0, p['W_u1'][:HV], p['W_u1'][HV:], dis0)
    h2 = _pallas_block(xw_y, srcA0, dstA0, dis0,
                       p['b_u1'], p['g_u1'], p['be_u1'])

    return _tc_mm(h2, p['W_out'], p['b_out'], act=False)
